# Initial kernel scaffold; baseline (speedup 1.0000x reference)
import jax
import jax.numpy as jnp
from jax.experimental import pallas as pl


def kernel(x, edge_index, weight_low):
    raise NotImplementedError("write your pallas kernel here")


# speedup vs baseline: 4.2780x; 4.2780x over previous
import functools

import jax
import jax.numpy as jnp
from jax import lax
from jax.experimental import pallas as pl
from jax.experimental.pallas import tpu as pltpu
from jax.experimental.pallas import tpu_sc as plsc

_N_NODES = 10000
_N_EDGES = 320000
_D = 128

_NC = 2
_NS = 16
_NW = _NC * _NS
_CHUNK = 128
_CHUNKS_PER_WORKER = 79
_EDGES_PER_WORKER = _CHUNKS_PER_WORKER * _CHUNK
_E_PAD = _EDGES_PER_WORKER * _NW
_ACC_ROWS = 10240
_ROWS_PER_TILE = _ACC_ROWS // _NS


def _sc_aggregate(x, src_p, dst_p):
    mesh = plsc.VectorSubcoreMesh(core_axis_name="c", subcore_axis_name="s")

    @functools.partial(
        pl.kernel,
        mesh=mesh,
        out_type=jax.ShapeDtypeStruct((_NC, _ACC_ROWS, _D), jnp.float32),
        scratch_types=[
            pltpu.VMEM_SHARED((_ACC_ROWS, _D), jnp.float32),
            pltpu.VMEM((_CHUNK,), jnp.int32),
            pltpu.VMEM((_CHUNK,), jnp.int32),
            pltpu.VMEM((_CHUNK, _D), jnp.float32),
            pltpu.SemaphoreType.DMA,
        ],
    )
    def sc_agg(x_hbm, src_hbm, dst_hbm, out_hbm, acc, sidx, didx, rows, sem):
        c = lax.axis_index("c")
        s = lax.axis_index("s")

        zero = jnp.zeros((16,), jnp.float32)

        def zrow(i, carry):
            for t in range(_D // 16):
                rows[i, pl.ds(t * 16, 16)] = zero
            return carry

        lax.fori_loop(0, _CHUNK, zrow, 0)

        for t in range(_ROWS_PER_TILE // _CHUNK):
            pltpu.sync_copy(
                rows, acc.at[pl.ds(s * _ROWS_PER_TILE + t * _CHUNK, _CHUNK)]
            )
        plsc.subcore_barrier()

        base = (c * _NS + s) * _EDGES_PER_WORKER

        def step(j, carry):
            off = pl.multiple_of(base + j * _CHUNK, _CHUNK)
            pltpu.sync_copy(src_hbm.at[pl.ds(off, _CHUNK)], sidx)
            pltpu.sync_copy(dst_hbm.at[pl.ds(off, _CHUNK)], didx)
            pltpu.async_copy(x_hbm.at[sidx], rows, sem).wait()
            pltpu.sync_copy(rows, acc.at[didx], add=True)
            return carry

        lax.fori_loop(0, _CHUNKS_PER_WORKER, step, 0)
        plsc.subcore_barrier()

        pltpu.sync_copy(
            acc.at[pl.ds(s * _ROWS_PER_TILE, _ROWS_PER_TILE)],
            out_hbm.at[c].at[pl.ds(s * _ROWS_PER_TILE, _ROWS_PER_TILE)],
        )

    return sc_agg(x, src_p, dst_p)


_BLK = 2000


def _tc_body(p_ref, w_ref, o_ref):
    s = p_ref[0] + p_ref[1]
    o_ref[...] = jnp.dot(s, w_ref[...], preferred_element_type=jnp.float32)


def _tc_combine(partials, w):
    return pl.pallas_call(
        _tc_body,
        grid=(_N_NODES // _BLK,),
        in_specs=[
            pl.BlockSpec((_NC, _BLK, _D), lambda i: (0, i, 0)),
            pl.BlockSpec((_D, _D), lambda i: (0, 0)),
        ],
        out_specs=pl.BlockSpec((_BLK, _D), lambda i: (i, 0)),
        out_shape=jax.ShapeDtypeStruct((_N_NODES, _D), jnp.float32),
    )(partials, w)


def kernel(x, edge_index, weight_low):
    src = edge_index[0]
    dst = edge_index[1]
    pad = _E_PAD - _N_EDGES
    src_p = jnp.concatenate([src, jnp.zeros((pad,), jnp.int32)])
    dst_p = jnp.concatenate([dst, jnp.full((pad,), _N_NODES, jnp.int32)])
    partials = _sc_aggregate(x, src_p, dst_p)
    return _tc_combine(partials, weight_low)
